# Initial kernel scaffold; baseline (speedup 1.0000x reference)
import jax
import jax.numpy as jnp
from jax.experimental import pallas as pl


def kernel(x, edge_index, edge_features, global_features, memory, params):
    raise NotImplementedError("write your pallas kernel here")


# speedup vs baseline: 2.9422x; 2.9422x over previous
import functools

import jax
import jax.numpy as jnp
from jax import lax
from jax.experimental import pallas as pl
from jax.experimental.pallas import tpu as pltpu
from jax.experimental.pallas import tpu_sc as plsc

NUM_MSG = 3
D = 128
_NW = 32
_SC_CHUNK = 512
_SCT_CHUNK = 256


def _mm(a, b):
    return lax.dot_general(a, b, (((1,), (0,)), ((), ())),
                           preferred_element_type=jnp.float32)


def _full(arr):
    return pl.BlockSpec(arr.shape, lambda i: (0,) * arr.ndim)


def _row(arr, nb):
    return pl.BlockSpec((nb, arr.shape[1]), lambda i: (i, 0))


def _slab_plan(rows2):
    slab = (-(-rows2 // _NW) + 7) // 8 * 8
    full = rows2 // slab
    rem = rows2 - full * slab
    return slab, full, rem


def _sc_gather(table, idx2):
    rows2 = idx2.shape[0]
    e = rows2 * 128
    slab, full, rem = _slab_plan(rows2)
    mesh = plsc.VectorSubcoreMesh(core_axis_name="c", subcore_axis_name="s")

    @functools.partial(
        pl.kernel,
        out_type=jax.ShapeDtypeStruct((e, D), jnp.float32),
        mesh=mesh,
        scratch_types=[
            pltpu.VMEM((slab, 128), jnp.int32),
            pltpu.VMEM((_SC_CHUNK, D), jnp.float32),
            pltpu.SemaphoreType.DMA,
        ],
    )
    def k(table_hbm, idx_hbm, out_hbm, idx_v, rows_v, sem):
        w = lax.axis_index("c") * 16 + lax.axis_index("s")

        @pl.when(w < full)
        def _():
            pltpu.sync_copy(idx_hbm.at[pl.ds(w * slab, slab)], idx_v)

        if rem:
            @pl.when(w == full)
            def _():
                pltpu.sync_copy(idx_hbm.at[pl.ds(full * slab, rem)],
                                idx_v.at[pl.ds(0, rem)])

        nch = jnp.where(w < full, slab // 4,
                        jnp.where(w == full, rem // 4, 0))

        def body(kk, carry):
            eb = w * slab * 128 + kk * _SC_CHUNK
            descs = [
                pltpu.async_copy(table_hbm.at[idx_v.at[kk * 4 + j]],
                                 rows_v.at[pl.ds(j * 128, 128)], sem)
                for j in range(4)
            ]
            for dsc in descs:
                dsc.wait()
            pltpu.sync_copy(rows_v, out_hbm.at[pl.ds(eb, _SC_CHUNK)])
            return carry

        lax.fori_loop(0, nch, body, 0)

    return k(table, idx2)


def _sc_scatter(vals, idx2, zeros):
    n = zeros.shape[0]
    rows2 = idx2.shape[0]
    slab, full, rem = _slab_plan(rows2)
    rpt = (n // 16) // 8 * 8
    tail = n - 16 * rpt
    mesh = plsc.VectorSubcoreMesh(core_axis_name="c", subcore_axis_name="s")

    @functools.partial(
        pl.kernel,
        out_type=jax.ShapeDtypeStruct((2 * n, D), jnp.float32),
        mesh=mesh,
        scratch_types=[
            pltpu.VMEM_SHARED((n, D), jnp.float32),
            pltpu.VMEM((slab, 128), jnp.int32),
            pltpu.VMEM((_SCT_CHUNK, D), jnp.float32),
        ],
    )
    def k(vals_hbm, idx_hbm, zeros_hbm, out_hbm, shared, idx_v, vals_v):
        c = lax.axis_index("c")
        s = lax.axis_index("s")
        w = c * 16 + s

        pltpu.sync_copy(zeros_hbm.at[pl.ds(s * rpt, rpt)],
                        shared.at[pl.ds(s * rpt, rpt)])
        if tail:
            @pl.when(s == 15)
            def _():
                pltpu.sync_copy(zeros_hbm.at[pl.ds(16 * rpt, tail)],
                                shared.at[pl.ds(16 * rpt, tail)])
        plsc.subcore_barrier()

        @pl.when(w < full)
        def _():
            pltpu.sync_copy(idx_hbm.at[pl.ds(w * slab, slab)], idx_v)

        if rem:
            @pl.when(w == full)
            def _():
                pltpu.sync_copy(idx_hbm.at[pl.ds(full * slab, rem)],
                                idx_v.at[pl.ds(0, rem)])

        nch = jnp.where(w < full, slab // 2,
                        jnp.where(w == full, rem // 2, 0))

        def body(kk, carry):
            eb = w * slab * 128 + kk * _SCT_CHUNK
            pltpu.sync_copy(vals_hbm.at[pl.ds(eb, _SCT_CHUNK)], vals_v)
            for j in range(2):
                pltpu.sync_copy(vals_v.at[pl.ds(j * 128, 128)],
                                shared.at[idx_v.at[kk * 2 + j]], add=True)
            return carry

        lax.fori_loop(0, nch, body, 0)
        plsc.subcore_barrier()

        pltpu.sync_copy(shared.at[pl.ds(s * rpt, rpt)],
                        out_hbm.at[pl.ds(c * n + s * rpt, rpt)])
        if tail:
            @pl.when(s == 15)
            def _():
                pltpu.sync_copy(shared.at[pl.ds(16 * rpt, tail)],
                                out_hbm.at[pl.ds(c * n + 16 * rpt, tail)])

    return k(vals, idx2, zeros)


def _encode_nodes(x, gf, p, nb):
    n = x.shape[0]

    def body(x_r, gf_r, w1_r, b1_r, w2_r, b2_r, gw1_r, gb1_r, gw2_r, gb2_r,
             xe_r):
        h = jnp.maximum(_mm(x_r[...], w1_r[...]) + b1_r[...], 0.0)
        h = jnp.maximum(_mm(h, w2_r[...]) + b2_r[...], 0.0)
        g = jnp.maximum(_mm(gf_r[...], gw1_r[...]) + gb1_r[...], 0.0)
        g = _mm(g, gw2_r[...]) + gb2_r[...]
        xe_r[...] = h + g

    args = (x, gf, p['ne_W1'], p['ne_b1'].reshape(1, -1), p['ne_W2'],
            p['ne_b2'].reshape(1, -1), p['ge_W1'], p['ge_b1'].reshape(1, -1),
            p['ge_W2'], p['ge_b2'].reshape(1, -1))
    specs = [_row(x, nb)] + [_full(a) for a in args[1:]]
    return pl.pallas_call(
        body, grid=(n // nb,), in_specs=specs,
        out_shape=jax.ShapeDtypeStruct((n, D), jnp.float32),
        out_specs=pl.BlockSpec((nb, D), lambda i: (i, 0)))(*args)


def _encode_edges(ef, p, be):
    e = ef.shape[0]

    def body(ef_r, w1_r, b1_r, w2_r, b2_r, out_r):
        h = jnp.maximum(_mm(ef_r[...], w1_r[...]) + b1_r[...], 0.0)
        out_r[...] = jnp.maximum(_mm(h, w2_r[...]) + b2_r[...], 0.0)

    args = (ef, p['ee_W1'], p['ee_b1'].reshape(1, -1), p['ee_W2'],
            p['ee_b2'].reshape(1, -1))
    specs = [_row(ef, be)] + [_full(a) for a in args[1:]]
    return pl.pallas_call(
        body, grid=(e // be,), in_specs=specs,
        out_shape=jax.ShapeDtypeStruct((e, D), jnp.float32),
        out_specs=pl.BlockSpec((be, D), lambda i: (i, 0)))(*args)


def _edge_msg(h_src, e_enc, w1, b1, w2, b2, be):
    e = h_src.shape[0]

    def body(g_r, e_r, w1_r, b1_r, w2_r, b2_r, out_r):
        cat = jnp.concatenate([g_r[...], e_r[...]], axis=1)
        h = jnp.maximum(_mm(cat, w1_r[...]) + b1_r[...], 0.0)
        out_r[...] = jnp.maximum(_mm(h, w2_r[...]) + b2_r[...], 0.0)

    args = (h_src, e_enc, w1, b1.reshape(1, -1), w2, b2.reshape(1, -1))
    specs = [_row(h_src, be), _row(e_enc, be)] + [_full(a) for a in args[2:]]
    return pl.pallas_call(
        body, grid=(e // be,), in_specs=specs,
        out_shape=jax.ShapeDtypeStruct((e, D), jnp.float32),
        out_specs=pl.BlockSpec((be, D), lambda i: (i, 0)))(*args)


def _update_nodes(xe, p0, p1, w1, b1, w2, b2, nb):
    n = xe.shape[0]

    def body(xe_r, p0_r, p1_r, w1_r, b1_r, w2_r, b2_r, xn_r):
        mv = p0_r[...] + p1_r[...]
        cat = jnp.concatenate([xe_r[...], mv], axis=1)
        h = jnp.maximum(_mm(cat, w1_r[...]) + b1_r[...], 0.0)
        h = jnp.maximum(_mm(h, w2_r[...]) + b2_r[...], 0.0)
        xn_r[...] = jnp.maximum(xe_r[...] + h, 0.0)

    args = (xe, p0, p1, w1, b1.reshape(1, -1), w2, b2.reshape(1, -1))
    specs = [_row(xe, nb), _row(p0, nb), _row(p1, nb)] + [_full(a) for a in args[3:]]
    return pl.pallas_call(
        body, grid=(n // nb,), in_specs=specs,
        out_shape=jax.ShapeDtypeStruct((n, D), jnp.float32),
        out_specs=pl.BlockSpec((nb, D), lambda i: (i, 0)))(*args)


def _gru_decode(xe, mem, p, nb):
    n = xe.shape[0]
    od = p['dec_W3'].shape[1]

    def body(xe_r, mem_r, wih_r, bih_r, whh_r, bhh_r, d1_r, db1_r, d2_r,
             db2_r, d3_r, db3_r, out_r, mem_out_r):
        gi = _mm(xe_r[...], wih_r[...]) + bih_r[...]
        gh = _mm(mem_r[...], whh_r[...]) + bhh_r[...]
        r = jax.nn.sigmoid(gi[:, :D] + gh[:, :D])
        z = jax.nn.sigmoid(gi[:, D:2 * D] + gh[:, D:2 * D])
        nn_ = jnp.tanh(gi[:, 2 * D:] + r * gh[:, 2 * D:])
        mem_new = (1.0 - z) * nn_ + z * mem_r[...]
        h = jnp.maximum(_mm(mem_new, d1_r[...]) + db1_r[...], 0.0)
        h = jnp.maximum(_mm(h, d2_r[...]) + db2_r[...], 0.0)
        out_r[...] = _mm(h, d3_r[...]) + db3_r[...]
        mem_out_r[...] = mem_new

    args = (xe, mem, p['gru_Wih'], p['gru_bih'].reshape(1, -1), p['gru_Whh'],
            p['gru_bhh'].reshape(1, -1), p['dec_W1'], p['dec_b1'].reshape(1, -1),
            p['dec_W2'], p['dec_b2'].reshape(1, -1), p['dec_W3'],
            p['dec_b3'].reshape(1, -1))
    specs = [_row(xe, nb), _row(mem, nb)] + [_full(a) for a in args[2:]]
    out_sh = (jax.ShapeDtypeStruct((n, od), jnp.float32),
              jax.ShapeDtypeStruct((n, D), jnp.float32))
    out_specs = (pl.BlockSpec((nb, od), lambda i: (i, 0)),
                 pl.BlockSpec((nb, D), lambda i: (i, 0)))
    return pl.pallas_call(body, grid=(n // nb,), in_specs=specs,
                          out_shape=out_sh, out_specs=out_specs)(*args)


def kernel(x, edge_index, edge_features, global_features, memory, params):
    p = params
    n = x.shape[0]
    e = edge_features.shape[0]
    nb = 2000 if n % 2000 == 0 else n
    be = 4000 if e % 4000 == 0 else e

    src2 = edge_index[0].astype(jnp.int32).reshape(e // 128, 128)
    tgt2 = edge_index[1].astype(jnp.int32).reshape(e // 128, 128)
    gf = global_features.reshape(1, -1)
    zeros = jnp.zeros((n, D), jnp.float32)

    xe = _encode_nodes(x, gf, p, nb)
    e_enc = _encode_edges(edge_features, p, be)

    for i in range(NUM_MSG):
        h_src = _sc_gather(xe, src2)
        m_ij = _edge_msg(h_src, e_enc, p['msg_W1'][i], p['msg_b1'][i],
                         p['msg_W2'][i], p['msg_b2'][i], be)
        partials = _sc_scatter(m_ij, tgt2, zeros)
        xe = _update_nodes(xe, partials[:n], partials[n:], p['upd_W1'][i],
                           p['upd_b1'][i], p['upd_W2'][i], p['upd_b2'][i], nb)

    return _gru_decode(xe, memory, p, nb)
